# Initial kernel scaffold; baseline (speedup 1.0000x reference)
import jax
import jax.numpy as jnp
from jax.experimental import pallas as pl


def kernel(inputs_embeds, attention_mask, token_type_ids):
    raise NotImplementedError("write your pallas kernel here")


# speedup vs baseline: 1.0270x; 1.0270x over previous
import functools

import jax
import jax.numpy as jnp
import numpy as np
from jax.experimental import pallas as pl

D_MODEL = 1024
SEQ_LEN = 2048
HALF = D_MODEL // 2

_PE_SPECS = (
    (4096, 2048, -1),
    (2048, 2048, -2),
    (4096, 2047, -1),
    (1024, 2048, -4),
    (2048, 2046, -2),
    (512, 2048, -8),
    (1024, 2044, -4),
)

_ROWS_PER_STEP = 256


def _pe_body(vals_ref, invf_ref, *out_refs, starts):
    i = pl.program_id(0)
    v = vals_ref[0, 0, :]
    invf = invf_ref[0, :]
    arg = v[:, None] * invf[None, :]
    s = jnp.sin(arg)
    c = jnp.cos(arg)
    for k, ref in enumerate(out_refs):
        lo, hi = starts[k], starts[k + 1]

        @pl.when((i >= lo) & (i < hi))
        def _():
            ref[:, :HALF] = s
            ref[:, HALF:] = c


def _build_pes(dtype):
    nblocks = [n // _ROWS_PER_STEP for (n, _, _) in _PE_SPECS]
    starts = [0]
    for nb in nblocks:
        starts.append(starts[-1] + nb)
    total_steps = starts[-1]

    vals = np.concatenate([
        first + step * np.arange(n, dtype=np.float32)
        for (n, first, step) in _PE_SPECS
    ]).reshape(total_steps, 1, _ROWS_PER_STEP)
    vals = jnp.asarray(vals, dtype=dtype)

    freq = jnp.arange(HALF, dtype=dtype)
    invf = (1.0 / (10000.0 ** (freq / HALF)))[None, :]

    out_shapes = [jax.ShapeDtypeStruct((n, D_MODEL), dtype) for (n, _, _) in _PE_SPECS]

    def out_map(k):
        lo, nb = starts[k], nblocks[k]
        return lambda i: (jnp.clip(i - lo, 0, nb - 1), 0)

    return pl.pallas_call(
        functools.partial(_pe_body, starts=tuple(starts)),
        grid=(total_steps,),
        in_specs=[
            pl.BlockSpec((1, 1, _ROWS_PER_STEP), lambda i: (i, 0, 0)),
            pl.BlockSpec((1, HALF), lambda i: (0, 0)),
        ],
        out_specs=[
            pl.BlockSpec((_ROWS_PER_STEP, D_MODEL), out_map(k))
            for k in range(len(_PE_SPECS))
        ],
        out_shape=out_shapes,
    )(vals, invf)


_TT_ROWS = 512


def _tt_body(row_ref, full_ref, ttm_ref, cls_ref):
    j = pl.program_id(0)
    b = pl.program_id(1)
    shape = (_TT_ROWS, SEQ_LEN)
    rows = jnp.broadcast_to(row_ref[0, 0, :][:, None], shape)
    cols = jnp.broadcast_to(full_ref[0, 0, :][None, :], shape)
    ttm_ref[0] = (rows == cols) | (rows == 2) | (cols == 2)

    @pl.when(b == 0)
    def _():
        ri = jax.lax.broadcasted_iota(jnp.int32, (_TT_ROWS, SEQ_LEN), 0)
        ci = jax.lax.broadcasted_iota(jnp.int32, (_TT_ROWS, SEQ_LEN), 1)
        cls_ref[...] = (((ri + j * _TT_ROWS) > 0) & (ci > 0)).astype(cls_ref.dtype)


def _build_ttm(token_type_ids, dtype):
    batch = token_type_ids.shape[0]
    ids3 = token_type_ids.reshape(batch, 1, SEQ_LEN)
    nj = SEQ_LEN // _TT_ROWS
    return pl.pallas_call(
        _tt_body,
        grid=(nj, batch),
        in_specs=[
            pl.BlockSpec((1, 1, _TT_ROWS), lambda j, b: (b, 0, j)),
            pl.BlockSpec((1, 1, SEQ_LEN), lambda j, b: (b, 0, 0)),
        ],
        out_specs=[
            pl.BlockSpec((1, _TT_ROWS, SEQ_LEN), lambda j, b: (b, j, 0)),
            pl.BlockSpec((_TT_ROWS, SEQ_LEN), lambda j, b: (j, 0)),
        ],
        out_shape=[
            jax.ShapeDtypeStruct((batch, SEQ_LEN, SEQ_LEN), jnp.bool_),
            jax.ShapeDtypeStruct((SEQ_LEN, SEQ_LEN), dtype),
        ],
    )(ids3, ids3)


def kernel(inputs_embeds, attention_mask, token_type_ids):
    dtype = inputs_embeds.dtype
    pes = _build_pes(dtype)
    ttm, cls_mask = _build_ttm(token_type_ids, dtype)
    return (*pes, ttm, attention_mask, cls_mask)


# speedup vs baseline: 1.7831x; 1.7363x over previous
import functools

import jax
import jax.numpy as jnp
import numpy as np
from jax.experimental import pallas as pl

D_MODEL = 1024
SEQ_LEN = 2048
HALF = D_MODEL // 2

_PE_SPECS = (
    (4096, 2048, -1),
    (2048, 2048, -2),
    (4096, 2047, -1),
    (1024, 2048, -4),
    (2048, 2046, -2),
    (512, 2048, -8),
    (1024, 2044, -4),
)

_ROWS_PER_STEP = 256


def _pe_body(vals_ref, invf_ref, *out_refs, starts):
    i = pl.program_id(0)
    v = vals_ref[0, 0, :]
    invf = invf_ref[0, :]
    arg8 = v[:8][:, None] * invf[None, :]
    s = jnp.sin(arg8)
    c = jnp.cos(arg8)
    d = v[1:2] - v[0:1]
    n = 8
    while n < _ROWS_PER_STEP:
        rot = (n * d)[:, None] * invf[None, :]
        rs = jnp.sin(rot)
        rc = jnp.cos(rot)
        s, c = (
            jnp.concatenate([s, s * rc + c * rs], axis=0),
            jnp.concatenate([c, c * rc - s * rs], axis=0),
        )
        n *= 2
    for k, ref in enumerate(out_refs):
        lo, hi = starts[k], starts[k + 1]

        @pl.when((i >= lo) & (i < hi))
        def _():
            ref[:, :HALF] = s
            ref[:, HALF:] = c


def _build_pes(dtype):
    nblocks = [n // _ROWS_PER_STEP for (n, _, _) in _PE_SPECS]
    starts = [0]
    for nb in nblocks:
        starts.append(starts[-1] + nb)
    total_steps = starts[-1]

    vals = np.concatenate([
        first + step * np.arange(n, dtype=np.float32)
        for (n, first, step) in _PE_SPECS
    ]).reshape(total_steps, 1, _ROWS_PER_STEP)
    vals = jnp.asarray(vals, dtype=dtype)

    freq = jnp.arange(HALF, dtype=dtype)
    invf = (1.0 / (10000.0 ** (freq / HALF)))[None, :]

    out_shapes = [jax.ShapeDtypeStruct((n, D_MODEL), dtype) for (n, _, _) in _PE_SPECS]

    def out_map(k):
        lo, nb = starts[k], nblocks[k]
        return lambda i: (jnp.clip(i - lo, 0, nb - 1), 0)

    return pl.pallas_call(
        functools.partial(_pe_body, starts=tuple(starts)),
        grid=(total_steps,),
        in_specs=[
            pl.BlockSpec((1, 1, _ROWS_PER_STEP), lambda i: (i, 0, 0)),
            pl.BlockSpec((1, HALF), lambda i: (0, 0)),
        ],
        out_specs=[
            pl.BlockSpec((_ROWS_PER_STEP, D_MODEL), out_map(k))
            for k in range(len(_PE_SPECS))
        ],
        out_shape=out_shapes,
    )(vals, invf)


_TT_ROWS = 512


def _tt_body(row_ref, full_ref, ttm_ref, cls_ref):
    j = pl.program_id(0)
    b = pl.program_id(1)
    shape = (_TT_ROWS, SEQ_LEN)
    rows = jnp.broadcast_to(row_ref[0, 0, :][:, None], shape)
    cols = jnp.broadcast_to(full_ref[0, 0, :][None, :], shape)
    ttm_ref[0] = (rows == cols) | (rows == 2) | (cols == 2)

    @pl.when(b == 0)
    def _():
        ri = jax.lax.broadcasted_iota(jnp.int32, (_TT_ROWS, SEQ_LEN), 0)
        ci = jax.lax.broadcasted_iota(jnp.int32, (_TT_ROWS, SEQ_LEN), 1)
        cls_ref[...] = (((ri + j * _TT_ROWS) > 0) & (ci > 0)).astype(cls_ref.dtype)


def _build_ttm(token_type_ids, dtype):
    batch = token_type_ids.shape[0]
    ids3 = token_type_ids.reshape(batch, 1, SEQ_LEN)
    nj = SEQ_LEN // _TT_ROWS
    return pl.pallas_call(
        _tt_body,
        grid=(nj, batch),
        in_specs=[
            pl.BlockSpec((1, 1, _TT_ROWS), lambda j, b: (b, 0, j)),
            pl.BlockSpec((1, 1, SEQ_LEN), lambda j, b: (b, 0, 0)),
        ],
        out_specs=[
            pl.BlockSpec((1, _TT_ROWS, SEQ_LEN), lambda j, b: (b, j, 0)),
            pl.BlockSpec((_TT_ROWS, SEQ_LEN), lambda j, b: (j, 0)),
        ],
        out_shape=[
            jax.ShapeDtypeStruct((batch, SEQ_LEN, SEQ_LEN), jnp.bool_),
            jax.ShapeDtypeStruct((SEQ_LEN, SEQ_LEN), dtype),
        ],
    )(ids3, ids3)


def kernel(inputs_embeds, attention_mask, token_type_ids):
    dtype = inputs_embeds.dtype
    pes = _build_pes(dtype)
    ttm, cls_mask = _build_ttm(token_type_ids, dtype)
    return (*pes, ttm, attention_mask, cls_mask)


# speedup vs baseline: 2.0791x; 1.1660x over previous
import functools

import jax
import jax.numpy as jnp
import numpy as np
from jax.experimental import pallas as pl

D_MODEL = 1024
SEQ_LEN = 2048
HALF = D_MODEL // 2

_PE_SPECS = (
    (4096, 2048, -1),
    (2048, 2048, -2),
    (4096, 2047, -1),
    (1024, 2048, -4),
    (2048, 2046, -2),
    (512, 2048, -8),
    (1024, 2044, -4),
)

_ROWS_PER_STEP = 512


def _pe_body(vals_ref, invf_ref, *out_refs, starts):
    i = pl.program_id(0)
    v = vals_ref[0, 0, :]
    invf = invf_ref[0, :]
    arg8 = v[:8][:, None] * invf[None, :]
    s8 = jnp.sin(arg8)
    c8 = jnp.cos(arg8)
    d = v[1:2] - v[0:1]
    rots = []
    n = 8
    while n < _ROWS_PER_STEP:
        rot = (n * d)[:, None] * invf[None, :]
        rots.append((n, jnp.sin(rot), jnp.cos(rot)))
        n *= 2
    for k, ref in enumerate(out_refs):
        lo, hi = starts[k], starts[k + 1]

        @pl.when((i >= lo) & (i < hi))
        def _():
            ref[0:8, :HALF] = s8
            ref[0:8, HALF:] = c8
            for n, rs, rc in rots:
                s = ref[0:n, :HALF]
                c = ref[0:n, HALF:]
                ref[n:2 * n, :HALF] = s * rc + c * rs
                ref[n:2 * n, HALF:] = c * rc - s * rs


def _build_pes(dtype):
    nblocks = [n // _ROWS_PER_STEP for (n, _, _) in _PE_SPECS]
    starts = [0]
    for nb in nblocks:
        starts.append(starts[-1] + nb)
    total_steps = starts[-1]

    vals = np.concatenate([
        first + step * np.arange(n, dtype=np.float32)
        for (n, first, step) in _PE_SPECS
    ]).reshape(total_steps, 1, _ROWS_PER_STEP)
    vals = jnp.asarray(vals, dtype=dtype)

    freq = jnp.arange(HALF, dtype=dtype)
    invf = (1.0 / (10000.0 ** (freq / HALF)))[None, :]

    out_shapes = [jax.ShapeDtypeStruct((n, D_MODEL), dtype) for (n, _, _) in _PE_SPECS]

    def out_map(k):
        lo, nb = starts[k], nblocks[k]
        return lambda i: (jnp.clip(i - lo, 0, nb - 1), 0)

    return pl.pallas_call(
        functools.partial(_pe_body, starts=tuple(starts)),
        grid=(total_steps,),
        in_specs=[
            pl.BlockSpec((1, 1, _ROWS_PER_STEP), lambda i: (i, 0, 0)),
            pl.BlockSpec((1, HALF), lambda i: (0, 0)),
        ],
        out_specs=[
            pl.BlockSpec((_ROWS_PER_STEP, D_MODEL), out_map(k))
            for k in range(len(_PE_SPECS))
        ],
        out_shape=out_shapes,
    )(vals, invf)


_TT_ROWS = 512


def _tt_body(row_ref, full_ref, ttm_ref, cls_ref):
    j = pl.program_id(0)
    b = pl.program_id(1)
    shape = (_TT_ROWS, SEQ_LEN)
    rows = jnp.broadcast_to(row_ref[0, 0, :][:, None], shape)
    cols = jnp.broadcast_to(full_ref[0, 0, :][None, :], shape)
    ttm_ref[0] = (rows == cols) | (rows == 2) | (cols == 2)

    @pl.when(b == 0)
    def _():
        ri = jax.lax.broadcasted_iota(jnp.int32, (_TT_ROWS, SEQ_LEN), 0)
        ci = jax.lax.broadcasted_iota(jnp.int32, (_TT_ROWS, SEQ_LEN), 1)
        cls_ref[...] = (((ri + j * _TT_ROWS) > 0) & (ci > 0)).astype(cls_ref.dtype)


def _build_ttm(token_type_ids, dtype):
    batch = token_type_ids.shape[0]
    ids3 = token_type_ids.reshape(batch, 1, SEQ_LEN)
    nj = SEQ_LEN // _TT_ROWS
    return pl.pallas_call(
        _tt_body,
        grid=(nj, batch),
        in_specs=[
            pl.BlockSpec((1, 1, _TT_ROWS), lambda j, b: (b, 0, j)),
            pl.BlockSpec((1, 1, SEQ_LEN), lambda j, b: (b, 0, 0)),
        ],
        out_specs=[
            pl.BlockSpec((1, _TT_ROWS, SEQ_LEN), lambda j, b: (b, j, 0)),
            pl.BlockSpec((_TT_ROWS, SEQ_LEN), lambda j, b: (j, 0)),
        ],
        out_shape=[
            jax.ShapeDtypeStruct((batch, SEQ_LEN, SEQ_LEN), jnp.bool_),
            jax.ShapeDtypeStruct((SEQ_LEN, SEQ_LEN), dtype),
        ],
    )(ids3, ids3)


def kernel(inputs_embeds, attention_mask, token_type_ids):
    dtype = inputs_embeds.dtype
    pes = _build_pes(dtype)
    ttm, cls_mask = _build_ttm(token_type_ids, dtype)
    return (*pes, ttm, attention_mask, cls_mask)
